# baseline (speedup 1.0000x reference)
import functools

import jax
import jax.numpy as jnp
from jax import lax
from jax.experimental import pallas as pl
from jax.experimental.pallas import tpu as pltpu
from jax.experimental.pallas import tpu_sc as plsc

DIM = 64
SCALE = 8.0
LANES = 16

NC = 2
NS = 16
NW = NC * NS

B_TOTAL = 4096 * 200
PER_W = B_TOTAL // NW
CHUNK = 128
NCHUNK = PER_W // CHUNK


def _emb_body(x_hbm, lut_hbm, out_hbm, idx_v, rows, sem):
    wid = lax.axis_index("s") * NC + lax.axis_index("c")
    pltpu.sync_copy(x_hbm.at[wid], idx_v)

    def chunk_body(g, carry):
        pltpu.async_copy(lut_hbm.at[idx_v.at[g]], rows, sem).wait()

        def row_body(i, c):
            for j in range(DIM // LANES):
                sl = (i, pl.ds(j * LANES, LANES))
                rows[sl] = rows[sl] * SCALE
            return c

        lax.fori_loop(0, CHUNK, row_body, 0)
        pltpu.sync_copy(rows, out_hbm.at[wid, g])
        return carry

    lax.fori_loop(0, NCHUNK, chunk_body, 0)


@functools.partial(
    pl.kernel,
    out_type=jax.ShapeDtypeStruct((NW, NCHUNK, CHUNK, DIM), jnp.float32),
    mesh=plsc.VectorSubcoreMesh(core_axis_name="c", subcore_axis_name="s"),
    compiler_params=pltpu.CompilerParams(use_tc_tiling_on_sc=False),
    scratch_types=[
        pltpu.VMEM((NCHUNK, CHUNK), jnp.int32),
        pltpu.VMEM((CHUNK, DIM), jnp.float32),
        pltpu.SemaphoreType.DMA,
    ],
)
def _emb(x_hbm, lut_hbm, out_hbm, idx_v, rows, sem):
    _emb_body(x_hbm, lut_hbm, out_hbm, idx_v, rows, sem)


def kernel(x, lut):
    n, s = x.shape
    xr = x.reshape(NW, NCHUNK, CHUNK).astype(jnp.int32)
    out = _emb(xr, lut)
    return out.reshape(n, s, DIM)


# speedup vs baseline: 1.2098x; 1.2098x over previous
import functools

import jax
import jax.numpy as jnp
from jax import lax
from jax.experimental import pallas as pl
from jax.experimental.pallas import tpu as pltpu
from jax.experimental.pallas import tpu_sc as plsc

DIM = 64
SCALE = 8.0
LANES = 16

NC = 2
NS = 16
NW = NC * NS

B_TOTAL = 4096 * 200
PER_W = B_TOTAL // NW
CHUNK = 128
NCHUNK = PER_W // CHUNK
NBUF = 8
ROWS_PER_ITER = 4


def _scale_buf(buf):
    def body(i, c):
        for r in range(ROWS_PER_ITER):
            for j in range(DIM // LANES):
                sl = (i * ROWS_PER_ITER + r, pl.ds(j * LANES, LANES))
                buf[sl] = buf[sl] * SCALE
        return c

    lax.fori_loop(0, CHUNK // ROWS_PER_ITER, body, 0)


def _emb_body(x_hbm, lut_hbm, out_hbm, idx_v, rows, gsem, ssem):
    wid = lax.axis_index("s") * NC + lax.axis_index("c")
    pltpu.sync_copy(x_hbm.at[wid], idx_v)

    for b in range(NBUF):
        pltpu.async_copy(lut_hbm.at[idx_v.at[b]], rows.at[b], gsem.at[b])

    def group(t, carry):
        g0 = t * NBUF
        for b in range(NBUF):
            c = g0 + b
            pltpu.make_async_copy(
                lut_hbm.at[idx_v.at[c]], rows.at[b], gsem.at[b]
            ).wait()
            _scale_buf(rows.at[b])
            pltpu.async_copy(rows.at[b], out_hbm.at[wid, c], ssem.at[b])
        for b in range(NBUF):
            c = g0 + b
            pltpu.make_async_copy(
                rows.at[b], out_hbm.at[wid, c], ssem.at[b]
            ).wait()
            pltpu.async_copy(
                lut_hbm.at[idx_v.at[c + NBUF]], rows.at[b], gsem.at[b]
            )
        return carry

    lax.fori_loop(0, NCHUNK // NBUF - 1, group, 0)

    g0 = NCHUNK - NBUF
    for b in range(NBUF):
        c = g0 + b
        pltpu.make_async_copy(
            lut_hbm.at[idx_v.at[c]], rows.at[b], gsem.at[b]
        ).wait()
        _scale_buf(rows.at[b])
        pltpu.async_copy(rows.at[b], out_hbm.at[wid, c], ssem.at[b])
    for b in range(NBUF):
        c = g0 + b
        pltpu.make_async_copy(
            rows.at[b], out_hbm.at[wid, c], ssem.at[b]
        ).wait()


@functools.partial(
    pl.kernel,
    out_type=jax.ShapeDtypeStruct((NW, NCHUNK, CHUNK, DIM), jnp.float32),
    mesh=plsc.VectorSubcoreMesh(core_axis_name="c", subcore_axis_name="s"),
    compiler_params=pltpu.CompilerParams(use_tc_tiling_on_sc=False),
    scratch_types=[
        pltpu.VMEM((NCHUNK, CHUNK), jnp.int32),
        pltpu.VMEM((NBUF, CHUNK, DIM), jnp.float32),
        pltpu.SemaphoreType.DMA((NBUF,)),
        pltpu.SemaphoreType.DMA((NBUF,)),
    ],
)
def _emb(x_hbm, lut_hbm, out_hbm, idx_v, rows, gsem, ssem):
    _emb_body(x_hbm, lut_hbm, out_hbm, idx_v, rows, gsem, ssem)


def kernel(x, lut):
    n, s = x.shape
    xr = x.reshape(NW, NCHUNK, CHUNK).astype(jnp.int32)
    out = _emb(xr, lut)
    return out.reshape(n, s, DIM)
